# baseline (speedup 1.0000x reference)
import jax
import jax.numpy as jnp
from jax.experimental import pallas as pl
from jax.experimental.pallas import tpu as pltpu

_D_BLOCK = 512


def _add_pe_kernel(x_ref, pe_ref, o_ref):
    o_ref[...] = x_ref[...] + pe_ref[...]


def kernel(x, position_embedding):
    B, S, D = x.shape
    bd = min(_D_BLOCK, D)
    grid = (D // bd, B)
    return pl.pallas_call(
        _add_pe_kernel,
        grid=grid,
        in_specs=[
            pl.BlockSpec((1, S, bd), lambda d, b: (b, 0, d)),
            pl.BlockSpec((S, bd), lambda d, b: (0, d)),
        ],
        out_specs=pl.BlockSpec((1, S, bd), lambda d, b: (b, 0, d)),
        out_shape=jax.ShapeDtypeStruct((B, S, D), x.dtype),
        compiler_params=pltpu.CompilerParams(
            dimension_semantics=("arbitrary", "arbitrary"),
        ),
    )(x, position_embedding)
